# Initial kernel scaffold; baseline (speedup 1.0000x reference)
import jax
import jax.numpy as jnp
from jax.experimental import pallas as pl


def kernel(h, edge_index, W, bias, norm):
    raise NotImplementedError("write your pallas kernel here")


# speedup vs baseline: 4.9175x; 4.9175x over previous
import functools

import jax
import jax.numpy as jnp
from jax import lax
from jax.experimental import pallas as pl
from jax.experimental.pallas import tpu as pltpu
from jax.experimental.pallas import tpu_sc as plsc

N_NODES = 10000
N_EDGES = 320000
F = 128

NC = 2
NS = 16
NW = NC * NS
EDGES_PER_TILE = N_EDGES // NW
CHUNK = 80
N_CHUNKS = EDGES_PER_TILE // CHUNK
RCH = 80
N_RCH = N_NODES // RCH
RCH_PER_TILE = -(-N_RCH // NS)


def _mm_body(h_ref, w_ref, norm_ref, o_ref):
    o_ref[...] = jnp.dot(h_ref[...], w_ref[...],
                         preferred_element_type=jnp.float32) * norm_ref[...]


def _epilogue_body(p_ref, norm_ref, bias_ref, o_ref):
    s = p_ref[0] + p_ref[1]
    v = s * norm_ref[...] + bias_ref[...]
    o_ref[...] = jnp.where(v >= 0, v, 0.2 * v)


def _seg_sum_body(m_hbm, src_hbm, dst_hbm, out_hbm,
                  src_v, dst_v, rows_v, zeros_v, accum_sh, sem):
    c = lax.axis_index("c")
    s = lax.axis_index("s")

    def zero_row(i, _):
        for j in range(F // 16):
            zeros_v[i, pl.ds(j * 16, 16)] = jnp.zeros((16,), jnp.float32)
        return 0
    lax.fori_loop(0, RCH, zero_row, 0)
    for k in range(RCH_PER_TILE):
        idx = s * RCH_PER_TILE + k

        @pl.when(idx < N_RCH)
        def _():
            pltpu.sync_copy(zeros_v, accum_sh.at[pl.ds(idx * RCH, RCH)])
    plsc.subcore_barrier()

    base = (c * NS + s) * EDGES_PER_TILE

    def chunk(i, _):
        off = base + i * CHUNK
        pltpu.sync_copy(src_hbm.at[pl.ds(off, CHUNK)], src_v)
        pltpu.sync_copy(dst_hbm.at[pl.ds(off, CHUNK)], dst_v)
        pltpu.async_copy(m_hbm.at[src_v], rows_v, sem).wait()
        pltpu.sync_copy(rows_v, accum_sh.at[dst_v], add=True)
        return 0
    lax.fori_loop(0, N_CHUNKS, chunk, 0)
    plsc.subcore_barrier()

    for k in range(RCH_PER_TILE):
        idx = s * RCH_PER_TILE + k

        @pl.when(idx < N_RCH)
        def _():
            pltpu.sync_copy(accum_sh.at[pl.ds(idx * RCH, RCH)],
                            out_hbm.at[c, pl.ds(idx * RCH, RCH)])


_seg_sum = functools.partial(
    pl.kernel,
    mesh=plsc.VectorSubcoreMesh(core_axis_name="c", subcore_axis_name="s"),
    out_type=jax.ShapeDtypeStruct((NC, N_NODES, F), jnp.float32),
    scratch_types=[
        pltpu.VMEM((CHUNK,), jnp.int32),
        pltpu.VMEM((CHUNK,), jnp.int32),
        pltpu.VMEM((CHUNK, F), jnp.float32),
        pltpu.VMEM((RCH, F), jnp.float32),
        pltpu.VMEM_SHARED((N_NODES, F), jnp.float32),
        pltpu.SemaphoreType.DMA,
    ],
)(_seg_sum_body)


@jax.jit
def kernel(h, edge_index, W, bias, norm):
    m = pl.pallas_call(
        _mm_body,
        grid=(10,),
        in_specs=[
            pl.BlockSpec((1000, F), lambda i: (i, 0)),
            pl.BlockSpec((F, F), lambda i: (0, 0)),
            pl.BlockSpec((1000, 1), lambda i: (i, 0)),
        ],
        out_specs=pl.BlockSpec((1000, F), lambda i: (i, 0)),
        out_shape=jax.ShapeDtypeStruct((N_NODES, F), jnp.float32),
    )(h, W, norm)

    partials = _seg_sum(m, edge_index[0], edge_index[1])

    out = pl.pallas_call(
        _epilogue_body,
        grid=(10,),
        in_specs=[
            pl.BlockSpec((NC, 1000, F), lambda i: (0, i, 0)),
            pl.BlockSpec((1000, 1), lambda i: (i, 0)),
            pl.BlockSpec((1, F), lambda i: (0, 0)),
        ],
        out_specs=pl.BlockSpec((1000, F), lambda i: (i, 0)),
        out_shape=jax.ShapeDtypeStruct((N_NODES, F), jnp.float32),
    )(partials, norm, bias.reshape(1, F))
    return out


# speedup vs baseline: 11.7630x; 2.3921x over previous
import functools

import jax
import jax.numpy as jnp
from jax import lax
from jax.experimental import pallas as pl
from jax.experimental.pallas import tpu as pltpu
from jax.experimental.pallas import tpu_sc as plsc

N_NODES = 10000
N_EDGES = 320000
F = 128

NC = 2
NS = 16
NW = NC * NS
EDGES_PER_TILE = N_EDGES // NW
CHUNK = 80
N_CHUNKS = EDGES_PER_TILE // CHUNK
NR = 3
NI = 4
RCH = 80
N_RCH = N_NODES // RCH
RCH_PER_TILE = -(-N_RCH // NS)


def _mm_body(h_ref, w_ref, norm_ref, o_ref):
    o_ref[...] = jnp.dot(h_ref[...], w_ref[...],
                         preferred_element_type=jnp.float32) * norm_ref[...]


def _epilogue_body(p_ref, norm_ref, bias_ref, o_ref):
    s = p_ref[0] + p_ref[1]
    v = s * norm_ref[...] + bias_ref[...]
    o_ref[...] = jnp.where(v >= 0, v, 0.2 * v)


def _seg_sum_body(m_hbm, src_hbm, dst_hbm, out_hbm,
                  src_bufs, dst_bufs, rows_v, accum_sh, gsem, ssem, isem):
    c = lax.axis_index("c")
    s = lax.axis_index("s")
    wid = c * NS + s
    ebase = wid * EDGES_PER_TILE

    def zero_row(i, _):
        for j in range(F // 16):
            rows_v[0, i, pl.ds(j * 16, 16)] = jnp.zeros((16,), jnp.float32)
        return 0
    lax.fori_loop(0, RCH, zero_row, 0)
    for k in range(RCH_PER_TILE):
        idx = s * RCH_PER_TILE + k

        @pl.when(idx < N_RCH)
        def _():
            pltpu.sync_copy(rows_v.at[0, pl.ds(0, RCH)],
                            accum_sh.at[pl.ds(idx * RCH, RCH)])
    plsc.subcore_barrier()

    def issue_idx(j):
        q = j % NI
        off = ebase + j * CHUNK
        pltpu.async_copy(src_hbm.at[pl.ds(off, CHUNK)], src_bufs.at[q],
                         isem.at[q])
        pltpu.async_copy(dst_hbm.at[pl.ds(off, CHUNK)], dst_bufs.at[q],
                         isem.at[q])

    def wait_idx(j):
        q = j % NI
        pltpu.make_async_copy(src_hbm.at[pl.ds(0, CHUNK)], src_bufs.at[q],
                              isem.at[q]).wait()
        pltpu.make_async_copy(src_hbm.at[pl.ds(0, CHUNK)], dst_bufs.at[q],
                              isem.at[q]).wait()

    def issue_gather(j):
        pltpu.async_copy(m_hbm.at[src_bufs.at[j % NI]],
                         rows_v.at[j % NR], gsem.at[j % NR])

    def wait_gather(j):
        pltpu.make_async_copy(m_hbm.at[pl.ds(0, CHUNK)], rows_v.at[j % NR],
                              gsem.at[j % NR]).wait()

    def issue_scatter(j):
        pltpu.async_copy(rows_v.at[j % NR], accum_sh.at[dst_bufs.at[j % NI]],
                         ssem.at[j % NR], add=True)

    def wait_scatter(j):
        pltpu.make_async_copy(rows_v.at[j % NR], accum_sh.at[pl.ds(0, CHUNK)],
                              ssem.at[j % NR]).wait()

    for j in range(NR):
        issue_idx(j)
    for j in range(NR - 1):
        wait_idx(j)
        issue_gather(j)

    def chunk_body(g, _):
        wait_gather(g)
        issue_scatter(g)
        h = g + NR - 1

        @pl.when((h < N_CHUNKS) & (g >= 1))
        def _():
            wait_scatter(h)

        @pl.when(h < N_CHUNKS)
        def _():
            wait_idx(h)
            issue_gather(h)

        @pl.when(g + NR < N_CHUNKS)
        def _():
            issue_idx(g + NR)
        return 0
    lax.fori_loop(0, N_CHUNKS, chunk_body, 0)
    for j in range(N_CHUNKS - NR, N_CHUNKS):
        wait_scatter(j)
    plsc.subcore_barrier()

    for k in range(RCH_PER_TILE):
        idx = s * RCH_PER_TILE + k

        @pl.when(idx < N_RCH)
        def _():
            pltpu.sync_copy(accum_sh.at[pl.ds(idx * RCH, RCH)],
                            out_hbm.at[c, pl.ds(idx * RCH, RCH)])


_seg_sum = functools.partial(
    pl.kernel,
    mesh=plsc.VectorSubcoreMesh(core_axis_name="c", subcore_axis_name="s"),
    out_type=jax.ShapeDtypeStruct((NC, N_NODES, F), jnp.float32),
    scratch_types=[
        pltpu.VMEM((NI, CHUNK), jnp.int32),
        pltpu.VMEM((NI, CHUNK), jnp.int32),
        pltpu.VMEM((NR, CHUNK, F), jnp.float32),
        pltpu.VMEM_SHARED((N_NODES, F), jnp.float32),
        pltpu.SemaphoreType.DMA((NR,)),
        pltpu.SemaphoreType.DMA((NR,)),
        pltpu.SemaphoreType.DMA((NI,)),
    ],
)(_seg_sum_body)


@jax.jit
def kernel(h, edge_index, W, bias, norm):
    m = pl.pallas_call(
        _mm_body,
        grid=(10,),
        in_specs=[
            pl.BlockSpec((1000, F), lambda i: (i, 0)),
            pl.BlockSpec((F, F), lambda i: (0, 0)),
            pl.BlockSpec((1000, 1), lambda i: (i, 0)),
        ],
        out_specs=pl.BlockSpec((1000, F), lambda i: (i, 0)),
        out_shape=jax.ShapeDtypeStruct((N_NODES, F), jnp.float32),
    )(h, W, norm)

    partials = _seg_sum(m, edge_index[0], edge_index[1])

    out = pl.pallas_call(
        _epilogue_body,
        grid=(10,),
        in_specs=[
            pl.BlockSpec((NC, 1000, F), lambda i: (0, i, 0)),
            pl.BlockSpec((1000, 1), lambda i: (i, 0)),
            pl.BlockSpec((1, F), lambda i: (0, 0)),
        ],
        out_specs=pl.BlockSpec((1000, F), lambda i: (i, 0)),
        out_shape=jax.ShapeDtypeStruct((N_NODES, F), jnp.float32),
    )(partials, norm, bias.reshape(1, F))
    return out


# speedup vs baseline: 12.1906x; 1.0364x over previous
import functools

import jax
import jax.numpy as jnp
from jax import lax
from jax.experimental import pallas as pl
from jax.experimental.pallas import tpu as pltpu
from jax.experimental.pallas import tpu_sc as plsc

N_NODES = 10000
N_EDGES = 320000
F = 128

NC = 2
NS = 16
NW = NC * NS
EDGES_PER_TILE = N_EDGES // NW
CHUNK = 80
N_CHUNKS = EDGES_PER_TILE // CHUNK
NR = 4
NI = 6
RCH = 80
N_RCH = N_NODES // RCH
RCH_PER_TILE = -(-N_RCH // NS)


def _mm_body(h_ref, w_ref, norm_ref, o_ref):
    o_ref[...] = jnp.dot(h_ref[...], w_ref[...],
                         preferred_element_type=jnp.float32) * norm_ref[...]


def _epilogue_body(p_ref, norm_ref, bias_ref, o_ref):
    s = p_ref[0] + p_ref[1]
    v = s * norm_ref[...] + bias_ref[...]
    o_ref[...] = jnp.where(v >= 0, v, 0.2 * v)


def _seg_sum_body(m_hbm, src_hbm, dst_hbm, out_hbm,
                  src_bufs, dst_bufs, rows_v, accum_sh, gsem, ssem, isem):
    c = lax.axis_index("c")
    s = lax.axis_index("s")
    wid = c * NS + s
    ebase = wid * EDGES_PER_TILE

    def issue_idx(j):
        q = j % NI
        off = ebase + j * CHUNK
        pltpu.async_copy(src_hbm.at[pl.ds(off, CHUNK)], src_bufs.at[q],
                         isem.at[q])
        pltpu.async_copy(dst_hbm.at[pl.ds(off, CHUNK)], dst_bufs.at[q],
                         isem.at[q])

    def wait_idx(j):
        q = j % NI
        pltpu.make_async_copy(src_hbm.at[pl.ds(0, CHUNK)], src_bufs.at[q],
                              isem.at[q]).wait()
        pltpu.make_async_copy(src_hbm.at[pl.ds(0, CHUNK)], dst_bufs.at[q],
                              isem.at[q]).wait()

    def issue_gather(j):
        pltpu.async_copy(m_hbm.at[src_bufs.at[j % NI]],
                         rows_v.at[j % NR], gsem.at[j % NR])

    def wait_gather(j):
        pltpu.make_async_copy(m_hbm.at[pl.ds(0, CHUNK)], rows_v.at[j % NR],
                              gsem.at[j % NR]).wait()

    def issue_scatter(j):
        pltpu.async_copy(rows_v.at[j % NR], accum_sh.at[dst_bufs.at[j % NI]],
                         ssem.at[j % NR], add=True)

    def wait_scatter(j):
        pltpu.make_async_copy(rows_v.at[j % NR], accum_sh.at[pl.ds(0, CHUNK)],
                              ssem.at[j % NR]).wait()

    for j in range(NR):
        issue_idx(j)

    def zero_row(i, _):
        for j in range(F // 16):
            rows_v[NR - 1, i, pl.ds(j * 16, 16)] = jnp.zeros((16,),
                                                             jnp.float32)
        return 0
    lax.fori_loop(0, RCH, zero_row, 0)
    for k in range(RCH_PER_TILE):
        idx = s * RCH_PER_TILE + k

        @pl.when(idx < N_RCH)
        def _():
            pltpu.sync_copy(rows_v.at[NR - 1, pl.ds(0, RCH)],
                            accum_sh.at[pl.ds(idx * RCH, RCH)])

    for j in range(NR - 1):
        wait_idx(j)
        issue_gather(j)
    plsc.subcore_barrier()

    def chunk_body(g, _):
        wait_gather(g)
        issue_scatter(g)
        h = g + NR - 1

        @pl.when((h < N_CHUNKS) & (g >= 1))
        def _():
            wait_scatter(h)

        @pl.when(h < N_CHUNKS)
        def _():
            wait_idx(h)
            issue_gather(h)

        @pl.when(g + NR < N_CHUNKS)
        def _():
            issue_idx(g + NR)
        return 0
    lax.fori_loop(0, N_CHUNKS, chunk_body, 0)
    for j in range(N_CHUNKS - NR, N_CHUNKS):
        wait_scatter(j)
    plsc.subcore_barrier()

    for k in range(RCH_PER_TILE):
        idx = s * RCH_PER_TILE + k

        @pl.when(idx < N_RCH)
        def _():
            pltpu.sync_copy(accum_sh.at[pl.ds(idx * RCH, RCH)],
                            out_hbm.at[c, pl.ds(idx * RCH, RCH)])


_seg_sum = functools.partial(
    pl.kernel,
    mesh=plsc.VectorSubcoreMesh(core_axis_name="c", subcore_axis_name="s"),
    out_type=jax.ShapeDtypeStruct((NC, N_NODES, F), jnp.float32),
    scratch_types=[
        pltpu.VMEM((NI, CHUNK), jnp.int32),
        pltpu.VMEM((NI, CHUNK), jnp.int32),
        pltpu.VMEM((NR, CHUNK, F), jnp.float32),
        pltpu.VMEM_SHARED((N_NODES, F), jnp.float32),
        pltpu.SemaphoreType.DMA((NR,)),
        pltpu.SemaphoreType.DMA((NR,)),
        pltpu.SemaphoreType.DMA((NI,)),
    ],
)(_seg_sum_body)


@jax.jit
def kernel(h, edge_index, W, bias, norm):
    m = pl.pallas_call(
        _mm_body,
        grid=(10,),
        in_specs=[
            pl.BlockSpec((1000, F), lambda i: (i, 0)),
            pl.BlockSpec((F, F), lambda i: (0, 0)),
            pl.BlockSpec((1000, 1), lambda i: (i, 0)),
        ],
        out_specs=pl.BlockSpec((1000, F), lambda i: (i, 0)),
        out_shape=jax.ShapeDtypeStruct((N_NODES, F), jnp.float32),
    )(h, W, norm)

    partials = _seg_sum(m, edge_index[0], edge_index[1])

    out = pl.pallas_call(
        _epilogue_body,
        grid=(10,),
        in_specs=[
            pl.BlockSpec((NC, 1000, F), lambda i: (0, i, 0)),
            pl.BlockSpec((1000, 1), lambda i: (i, 0)),
            pl.BlockSpec((1, F), lambda i: (0, 0)),
        ],
        out_specs=pl.BlockSpec((1000, F), lambda i: (i, 0)),
        out_shape=jax.ShapeDtypeStruct((N_NODES, F), jnp.float32),
    )(partials, norm, bias.reshape(1, F))
    return out


# speedup vs baseline: 13.7850x; 1.1308x over previous
import functools

import jax
import jax.numpy as jnp
from jax import lax
from jax.experimental import pallas as pl
from jax.experimental.pallas import tpu as pltpu
from jax.experimental.pallas import tpu_sc as plsc

N_NODES = 10000
N_EDGES = 320000
F = 128

NC = 2
NS = 16
NW = NC * NS
EDGES_PER_TILE = N_EDGES // NW
CHUNK = 80
N_CHUNKS = EDGES_PER_TILE // CHUNK
NR = 4
NI = 6
RCH = 80
N_RCH = N_NODES // RCH
RCH_PER_TILE = -(-N_RCH // NS)


def _mm_body(h_ref, w_ref, norm_ref, o_ref):
    o_ref[...] = jnp.dot(h_ref[...], w_ref[...],
                         preferred_element_type=jnp.float32) * norm_ref[...]


def _epilogue_body(p_ref, norm_ref, bias_ref, o_ref):
    s = p_ref[0] + p_ref[1]
    v = s * norm_ref[...] + bias_ref[...]
    o_ref[...] = jnp.where(v >= 0, v, 0.2 * v)


def _seg_sum_body(m_hbm, ei_hbm, out_hbm,
                  src_bufs, dst_bufs, rows_v, accum_sh, gsem, ssem, isem):
    c = lax.axis_index("c")
    s = lax.axis_index("s")
    wid = c * NS + s
    ebase = wid * EDGES_PER_TILE

    def issue_idx(j):
        q = j % NI
        off = ebase + j * CHUNK
        pltpu.async_copy(ei_hbm.at[pl.ds(off, CHUNK)], src_bufs.at[q],
                         isem.at[q])
        pltpu.async_copy(ei_hbm.at[pl.ds(N_EDGES + off, CHUNK)],
                         dst_bufs.at[q], isem.at[q])

    def wait_idx(j):
        q = j % NI
        pltpu.make_async_copy(ei_hbm.at[pl.ds(0, CHUNK)], src_bufs.at[q],
                              isem.at[q]).wait()
        pltpu.make_async_copy(ei_hbm.at[pl.ds(0, CHUNK)], dst_bufs.at[q],
                              isem.at[q]).wait()

    def issue_gather(j):
        pltpu.async_copy(m_hbm.at[src_bufs.at[j % NI]],
                         rows_v.at[j % NR], gsem.at[j % NR])

    def wait_gather(j):
        pltpu.make_async_copy(m_hbm.at[pl.ds(0, CHUNK)], rows_v.at[j % NR],
                              gsem.at[j % NR]).wait()

    def issue_scatter(j):
        pltpu.async_copy(rows_v.at[j % NR], accum_sh.at[dst_bufs.at[j % NI]],
                         ssem.at[j % NR], add=True)

    def wait_scatter(j):
        pltpu.make_async_copy(rows_v.at[j % NR], accum_sh.at[pl.ds(0, CHUNK)],
                              ssem.at[j % NR]).wait()

    for j in range(NR):
        issue_idx(j)

    def zero_row(i, _):
        for j in range(F // 16):
            rows_v[NR - 1, i, pl.ds(j * 16, 16)] = jnp.zeros((16,),
                                                             jnp.float32)
        return 0
    lax.fori_loop(0, RCH, zero_row, 0)
    for k in range(RCH_PER_TILE):
        idx = s * RCH_PER_TILE + k

        @pl.when(idx < N_RCH)
        def _():
            pltpu.sync_copy(rows_v.at[NR - 1, pl.ds(0, RCH)],
                            accum_sh.at[pl.ds(idx * RCH, RCH)])

    for j in range(NR - 1):
        wait_idx(j)
        issue_gather(j)
    plsc.subcore_barrier()

    def chunk_body(g, _):
        wait_gather(g)
        issue_scatter(g)
        h = g + NR - 1

        @pl.when((h < N_CHUNKS) & (g >= 1))
        def _():
            wait_scatter(h)

        @pl.when(h < N_CHUNKS)
        def _():
            wait_idx(h)
            issue_gather(h)

        @pl.when(g + NR < N_CHUNKS)
        def _():
            issue_idx(g + NR)
        return 0
    lax.fori_loop(0, N_CHUNKS, chunk_body, 0)
    for j in range(N_CHUNKS - NR, N_CHUNKS):
        wait_scatter(j)
    plsc.subcore_barrier()

    for k in range(RCH_PER_TILE):
        idx = s * RCH_PER_TILE + k

        @pl.when(idx < N_RCH)
        def _():
            pltpu.sync_copy(accum_sh.at[pl.ds(idx * RCH, RCH)],
                            out_hbm.at[c, pl.ds(idx * RCH, RCH)])


_seg_sum = functools.partial(
    pl.kernel,
    mesh=plsc.VectorSubcoreMesh(core_axis_name="c", subcore_axis_name="s"),
    out_type=jax.ShapeDtypeStruct((NC, N_NODES, F), jnp.float32),
    scratch_types=[
        pltpu.VMEM((NI, CHUNK), jnp.int32),
        pltpu.VMEM((NI, CHUNK), jnp.int32),
        pltpu.VMEM((NR, CHUNK, F), jnp.float32),
        pltpu.VMEM_SHARED((N_NODES, F), jnp.float32),
        pltpu.SemaphoreType.DMA((NR,)),
        pltpu.SemaphoreType.DMA((NR,)),
        pltpu.SemaphoreType.DMA((NI,)),
    ],
)(_seg_sum_body)


@jax.jit
def kernel(h, edge_index, W, bias, norm):
    m = pl.pallas_call(
        _mm_body,
        grid=(2,),
        in_specs=[
            pl.BlockSpec((5000, F), lambda i: (i, 0)),
            pl.BlockSpec((F, F), lambda i: (0, 0)),
            pl.BlockSpec((5000, 1), lambda i: (i, 0)),
        ],
        out_specs=pl.BlockSpec((5000, F), lambda i: (i, 0)),
        out_shape=jax.ShapeDtypeStruct((N_NODES, F), jnp.float32),
    )(h, W, norm)

    partials = _seg_sum(m, edge_index.reshape(2 * N_EDGES))

    out = pl.pallas_call(
        _epilogue_body,
        grid=(2,),
        in_specs=[
            pl.BlockSpec((NC, 5000, F), lambda i: (0, i, 0)),
            pl.BlockSpec((5000, 1), lambda i: (i, 0)),
            pl.BlockSpec((1, F), lambda i: (0, 0)),
        ],
        out_specs=pl.BlockSpec((5000, F), lambda i: (i, 0)),
        out_shape=jax.ShapeDtypeStruct((N_NODES, F), jnp.float32),
    )(partials, norm, bias.reshape(1, F))
    return out


# speedup vs baseline: 13.9836x; 1.0144x over previous
import functools

import jax
import jax.numpy as jnp
from jax import lax
from jax.experimental import pallas as pl
from jax.experimental.pallas import tpu as pltpu
from jax.experimental.pallas import tpu_sc as plsc

N_NODES = 10000
N_EDGES = 320000
F = 128

NC = 2
NS = 16
NW = NC * NS
CHUNK = 128
N_CHUNKS = N_EDGES // CHUNK
CH_LO = N_CHUNKS // NW
N_HI = N_CHUNKS - CH_LO * NW
NR = 3
NI = 4
RCH = 80
N_RCH = N_NODES // RCH
RCH_PER_TILE = -(-N_RCH // NS)


def _mm_body(h_ref, w_ref, norm_ref, o_ref):
    o_ref[...] = jnp.dot(h_ref[...], w_ref[...],
                         preferred_element_type=jnp.float32) * norm_ref[...]


def _epilogue_body(p_ref, norm_ref, bias_ref, o_ref):
    s = p_ref[0] + p_ref[1]
    v = s * norm_ref[...] + bias_ref[...]
    o_ref[...] = jnp.where(v >= 0, v, 0.2 * v)


def _seg_sum_body(m_hbm, ei_hbm, out_hbm,
                  ibuf, rows_v, accum_sh, gsem, ssem, isem):
    c = lax.axis_index("c")
    s = lax.axis_index("s")
    wid = c * NS + s
    cstart = CH_LO * wid + jnp.minimum(wid, N_HI)
    nch = CH_LO + jnp.where(wid < N_HI, 1, 0)

    def issue_idx(j):
        q = j % NI
        off = (cstart + j) * CHUNK
        pltpu.async_copy(ei_hbm.at[pl.ds(0, 2), pl.ds(off, CHUNK)],
                         ibuf.at[q], isem.at[q])

    def wait_idx(j):
        q = j % NI
        pltpu.make_async_copy(ei_hbm.at[pl.ds(0, 2), pl.ds(0, CHUNK)],
                              ibuf.at[q], isem.at[q]).wait()

    def issue_gather(j):
        pltpu.async_copy(m_hbm.at[ibuf.at[j % NI, 0]],
                         rows_v.at[j % NR], gsem.at[j % NR])

    def wait_gather(j):
        pltpu.make_async_copy(m_hbm.at[pl.ds(0, CHUNK)], rows_v.at[j % NR],
                              gsem.at[j % NR]).wait()

    def issue_scatter(j):
        pltpu.async_copy(rows_v.at[j % NR], accum_sh.at[ibuf.at[j % NI, 1]],
                         ssem.at[j % NR], add=True)

    def wait_scatter(j):
        pltpu.make_async_copy(rows_v.at[j % NR], accum_sh.at[pl.ds(0, CHUNK)],
                              ssem.at[j % NR]).wait()

    for j in range(NR):
        issue_idx(j)

    def zero_row(i, _):
        for j in range(F // 16):
            rows_v[NR - 1, i, pl.ds(j * 16, 16)] = jnp.zeros((16,),
                                                             jnp.float32)
        return 0
    lax.fori_loop(0, RCH, zero_row, 0)
    for k in range(RCH_PER_TILE):
        idx = s * RCH_PER_TILE + k

        @pl.when(idx < N_RCH)
        def _():
            pltpu.sync_copy(rows_v.at[NR - 1, pl.ds(0, RCH)],
                            accum_sh.at[pl.ds(idx * RCH, RCH)])

    for j in range(NR - 1):
        wait_idx(j)
        issue_gather(j)
    plsc.subcore_barrier()

    def chunk_body(g, _):
        wait_gather(g)
        issue_scatter(g)
        h = g + NR - 1

        @pl.when((h < nch) & (g >= 1))
        def _():
            wait_scatter(h)

        @pl.when(h < nch)
        def _():
            wait_idx(h)
            issue_gather(h)

        @pl.when(g + NR < nch)
        def _():
            issue_idx(g + NR)
        return 0
    lax.fori_loop(0, nch, chunk_body, 0)

    def drain(j, _):
        wait_scatter(j)
        return 0
    lax.fori_loop(nch - NR, nch, drain, 0)
    plsc.subcore_barrier()

    for k in range(RCH_PER_TILE):
        idx = s * RCH_PER_TILE + k

        @pl.when(idx < N_RCH)
        def _():
            pltpu.sync_copy(accum_sh.at[pl.ds(idx * RCH, RCH)],
                            out_hbm.at[c, pl.ds(idx * RCH, RCH)])


_seg_sum = functools.partial(
    pl.kernel,
    mesh=plsc.VectorSubcoreMesh(core_axis_name="c", subcore_axis_name="s"),
    out_type=jax.ShapeDtypeStruct((NC, N_NODES, F), jnp.float32),
    scratch_types=[
        pltpu.VMEM((NI, 2, CHUNK), jnp.int32),
        pltpu.VMEM((NR, CHUNK, F), jnp.float32),
        pltpu.VMEM_SHARED((N_NODES, F), jnp.float32),
        pltpu.SemaphoreType.DMA((NR,)),
        pltpu.SemaphoreType.DMA((NR,)),
        pltpu.SemaphoreType.DMA((NI,)),
    ],
)(_seg_sum_body)


@jax.jit
def kernel(h, edge_index, W, bias, norm):
    m = pl.pallas_call(
        _mm_body,
        grid=(2,),
        in_specs=[
            pl.BlockSpec((5000, F), lambda i: (i, 0)),
            pl.BlockSpec((F, F), lambda i: (0, 0)),
            pl.BlockSpec((5000, 1), lambda i: (i, 0)),
        ],
        out_specs=pl.BlockSpec((5000, F), lambda i: (i, 0)),
        out_shape=jax.ShapeDtypeStruct((N_NODES, F), jnp.float32),
    )(h, W, norm)

    partials = _seg_sum(m, edge_index)

    out = pl.pallas_call(
        _epilogue_body,
        grid=(2,),
        in_specs=[
            pl.BlockSpec((NC, 5000, F), lambda i: (0, i, 0)),
            pl.BlockSpec((5000, 1), lambda i: (i, 0)),
            pl.BlockSpec((1, F), lambda i: (0, 0)),
        ],
        out_specs=pl.BlockSpec((5000, F), lambda i: (i, 0)),
        out_shape=jax.ShapeDtypeStruct((N_NODES, F), jnp.float32),
    )(partials, norm, bias.reshape(1, F))
    return out
